# baseline (speedup 1.0000x reference)
import functools

import jax
import jax.numpy as jnp
from jax import lax
from jax.experimental import pallas as pl
from jax.experimental.pallas import tpu as pltpu
from jax.experimental.pallas import tpu_sc as plsc

_B = 256
_SIZE = 25
_N = _SIZE * _SIZE
_STRIDE = 8.0
_ORI = -96.0
_NW = 32
_RPW = _B // _NW


def _div25(p):
    return (p * 1311) >> 15


def _polar_pick_sc(cls, loc):
    mesh = plsc.VectorSubcoreMesh(core_axis_name="c", subcore_axis_name="s")

    @functools.partial(
        pl.kernel,
        mesh=mesh,
        out_type=jax.ShapeDtypeStruct((_B * 2,), jnp.float32),
        compiler_params=pltpu.CompilerParams(
            needs_layout_passes=False, use_tc_tiling_on_sc=False),
        scratch_types=[
            pltpu.VMEM((_RPW, 2, _SIZE, _SIZE), jnp.float32),
            pltpu.VMEM((_RPW, 4, _SIZE, _SIZE), jnp.float32),
            pltpu.VMEM((16,), jnp.float32),
            pltpu.SemaphoreType.DMA,
        ],
    )
    def k(cls_hbm, loc_hbm, out_hbm, cls_v, loc_v, out_v, sem):
        c = lax.axis_index("c")
        s = lax.axis_index("s")
        w = s * 2 + c
        base = w * _RPW
        loc_cp = pltpu.async_copy(loc_hbm.at[pl.ds(base, _RPW)], loc_v, sem)
        pltpu.sync_copy(cls_hbm.at[pl.ds(base, _RPW)], cls_v)

        lane = lax.iota(jnp.int32, 16)
        row_l = lane >> 1
        quad_l = lane >> 2
        one = jnp.full((16,), 1, jnp.int32)

        def _allreduce(v, binop):
            for step in (1, 2, 4, 8):
                shuf = v.at[lane ^ step].get(mode="promise_in_bounds")
                v = binop(v, shuf)
            return v

        def _pick(v, pos):
            return v.at[pos].get(mode="promise_in_bounds")

        def _scan_rows(rows):
            idx_quad = jnp.zeros((16,), jnp.int32)
            for j, r in enumerate(rows):
                rsplat = jnp.full((16,), r, jnp.int32)
                vmax = None
                for chunk in range(40):
                    p = jnp.minimum(lane + chunk * 16, jnp.int32(_N - 1))
                    y = _div25(p)
                    x = p - y * _SIZE
                    v = plsc.load_gather(cls_v, [rsplat, one, y, x])
                    if vmax is None:
                        vmax, vidx = v, p
                    else:
                        gt = v > vmax
                        vmax = jnp.maximum(vmax, v)
                        vidx = jnp.where(gt, p, vidx)
                m = _allreduce(vmax, jnp.maximum)
                cand = jnp.where(vmax == m, vidx, jnp.int32(2**30))
                idx_vec = _allreduce(cand, jnp.minimum)
                idx_quad = jnp.where(quad_l == j, idx_vec, idx_quad)
            return idx_quad

        iq0 = _scan_rows(range(0, 4))
        iq1 = _scan_rows(range(4, 8))

        kbit = lane & 1
        pairpos = ((row_l & 3) << 2) | kbit
        idx_pair = jnp.where(lane < 8, _pick(iq0, pairpos), _pick(iq1, pairpos))
        ypair = _div25(idx_pair)
        xpair = idx_pair - ypair * _SIZE
        sel = jnp.where(kbit == 0, xpair, ypair)
        p = sel.astype(jnp.float32) * jnp.float32(_STRIDE) + jnp.float32(_ORI)

        loc_cp.wait()
        g1 = plsc.load_gather(loc_v, [row_l, kbit, ypair, xpair])
        g2 = plsc.load_gather(loc_v, [row_l, kbit + 2, ypair, xpair])
        out_v[...] = p + (g2 - g1) * jnp.float32(0.5)
        pltpu.sync_copy(out_v, out_hbm.at[pl.ds(base * 2, 16)])

    return k(cls, loc)


def kernel(cls, loc):
    return _polar_pick_sc(cls, loc).reshape(_B, 2)
